# baseline (speedup 1.0000x reference)
import functools

import jax
import jax.numpy as jnp
from jax import lax
from jax.experimental import pallas as pl
from jax.experimental.pallas import tpu as pltpu
from jax.experimental.pallas import tpu_sc as plsc

LEVELS = 4
N = 4
C = 128
H = 128
W = 128
R = 6272
RB = 784
J = R // RB

NSUB = 16
NCORE = 2
CHALF = C // NCORE
NV = CHALF // 16
HW = H * W
RPT = HW // NSUB
GR = 32
CH = 4 * GR
G_PER = R // GR
NG = LEVELS * G_PER
GPT = NG // NSUB


def _weights_body(x_ref, y_ref, w_ref, idx_ref):
    x = x_ref[...]
    y = y_ref[...]
    fx = jnp.floor(x)
    fy = jnp.floor(y)
    xp = x - fx
    yp = y - fy
    vx = xp * xp + (1.0 - xp) * (1.0 - xp)
    vy = yp * yp + (1.0 - yp) * (1.0 - yp)
    invq = 0.25 / (vx * vy)
    fxi = fx.astype(jnp.int32)
    fyi = fy.astype(jnp.int32)
    cxi = fxi + 1
    cyi = fyi + 1
    u0 = (1.0 - xp) * invq
    u1 = xp * invq
    v0 = 1.0 - yp
    v1 = yp
    bx0 = fxi < H
    bx1 = cxi < H
    by0 = fyi < H
    by1 = cyi < H
    zf = jnp.zeros_like(x)
    zi = jnp.zeros_like(fxi)
    m11 = bx0 & by0
    m12 = bx0 & by1
    m21 = bx1 & by0
    m22 = bx1 & by1
    w_ref[0, :, :] = jnp.where(m11, u0 * v0, zf)
    w_ref[1, :, :] = jnp.where(m12, u0 * v1, zf)
    w_ref[2, :, :] = jnp.where(m21, u1 * v0, zf)
    w_ref[3, :, :] = jnp.where(m22, u1 * v1, zf)
    idx_ref[0, :, :] = jnp.where(m11, fxi * W + fyi, zi)
    idx_ref[1, :, :] = jnp.where(m12, fxi * W + cyi, zi)
    idx_ref[2, :, :] = jnp.where(m21, cxi * W + fyi, zi)
    idx_ref[3, :, :] = jnp.where(m22, cxi * W + cyi, zi)


def _phase_a(arc):
    return pl.pallas_call(
        _weights_body,
        grid=(N, LEVELS, J),
        in_specs=[
            pl.BlockSpec((None, None, RB, 1),
                         lambda n, l, j: (l, n, j, 0)),
            pl.BlockSpec((None, None, RB, 1),
                         lambda n, l, j: (l, n, j, 0)),
        ],
        out_specs=[
            pl.BlockSpec((None, None, 4, RB, 1),
                         lambda n, l, j: (n, l, 0, j, 0)),
            pl.BlockSpec((None, None, 4, RB, 1),
                         lambda n, l, j: (n, l, 0, j, 0)),
        ],
        out_shape=[
            jax.ShapeDtypeStruct((N, LEVELS, 4, R, 1), jnp.float32),
            jax.ShapeDtypeStruct((N, LEVELS, 4, R, 1), jnp.int32),
        ],
    )(arc[:, 0], arc[:, 1])


def _sc_body(fr_hbm, w_hbm, idx_hbm, zeros_hbm, out_hbm,
             idx_all, w_all, fbufs, rbufs, fsems, ssems, acc):
    cid = lax.axis_index("c")
    sid = lax.axis_index("s")
    col0 = cid * CHALF

    for n in range(N):
        pltpu.sync_copy(zeros_hbm, acc.at[pl.ds(sid * RPT, RPT)])
        pltpu.sync_copy(idx_hbm.at[n, pl.ds(sid * GPT, GPT)], idx_all)
        pltpu.sync_copy(w_hbm.at[n, pl.ds(sid * GPT, GPT)], w_all)
        plsc.subcore_barrier()
        gbase = sid * GPT

        def feat_start(k, s):
            g = jnp.minimum(gbase + k, NG - 1)
            l = g // G_PER
            r0 = (g - l * G_PER) * GR
            pltpu.async_copy(
                fr_hbm.at[l, n, pl.ds(r0, GR), pl.ds(col0, CHALF)],
                fbufs[s], fsems[s])

        def feat_wait(s):
            pltpu.make_async_copy(
                fr_hbm.at[0, n, pl.ds(0, GR), pl.ds(col0, CHALF)],
                fbufs[s], fsems[s]).wait()

        def compute(k, s):
            fb = fbufs[s]
            rb = rbufs[s]

            def half_body(h, _):
                j0 = 16 * h
                wv = [w_all[k, pl.ds(q * GR + j0, 16)] for q in range(4)]
                for j in range(16):
                    fv = [fb[j0 + j, pl.ds(16 * c, 16)] for c in range(NV)]
                    for q in range(4):
                        wsc = wv[q][j]
                        for c in range(NV):
                            rb[q * GR + j0 + j, pl.ds(16 * c, 16)] = (
                                wsc * fv[c])
                return _

            lax.fori_loop(0, GR // 16, half_body, None)

        def scat_start(k, s):
            pltpu.async_copy(rbufs[s], acc.at[idx_all.at[k]], ssems[s],
                             add=True)

        def scat_wait(k, s):
            pltpu.make_async_copy(rbufs[s], acc.at[idx_all.at[k]],
                                  ssems[s]).wait()

        feat_start(jnp.int32(0), 0)
        feat_start(jnp.int32(1), 1)
        for k0 in range(2):
            k = jnp.int32(k0)
            feat_wait(k0)
            compute(k, k0)
            scat_start(k, k0)
            feat_start(k + 2, k0)

        def pair_body(i, _):
            for s in range(2):
                k = 2 * i + 2 + s
                feat_wait(s)
                scat_wait(k - 2, s)
                compute(k, s)
                scat_start(k, s)
                feat_start(k + 2, s)
            return _

        lax.fori_loop(0, 23, pair_body, None)
        k = jnp.int32(GPT - 1)
        feat_wait(0)
        scat_wait(k - 2, 0)
        compute(k, 0)
        scat_start(k, 0)
        feat_wait(1)
        scat_wait(k - 1, 1)
        scat_wait(k, 0)
        plsc.subcore_barrier()
        pltpu.sync_copy(
            acc.at[pl.ds(sid * RPT, RPT)],
            out_hbm.at[n, pl.ds(sid * RPT, RPT), pl.ds(col0, CHALF)])
        plsc.subcore_barrier()


def _phase_b(fr, wg, idxg, zeros):
    mesh = plsc.VectorSubcoreMesh(core_axis_name="c", subcore_axis_name="s")
    f = pl.kernel(
        _sc_body,
        out_type=jax.ShapeDtypeStruct((N, HW, C), jnp.float32),
        mesh=mesh,
        scratch_types=[
            pltpu.VMEM((GPT, CH), jnp.int32),
            pltpu.VMEM((GPT, CH), jnp.float32),
            tuple(pltpu.VMEM((GR, CHALF), jnp.float32) for _ in range(2)),
            tuple(pltpu.VMEM((CH, CHALF), jnp.float32) for _ in range(2)),
            tuple(pltpu.SemaphoreType.DMA for _ in range(2)),
            tuple(pltpu.SemaphoreType.DMA for _ in range(2)),
            pltpu.VMEM_SHARED((HW, CHALF), jnp.float32),
        ],
        compiler_params=pltpu.CompilerParams(use_tc_tiling_on_sc=False),
    )
    return f(fr, wg, idxg, zeros)


def _transpose_body(in_ref, out_ref):
    out_ref[...] = in_ref[...].T


def _phase_c(acc):
    BLK = 1024
    return pl.pallas_call(
        _transpose_body,
        grid=(N, HW // BLK),
        in_specs=[pl.BlockSpec((None, BLK, C), lambda n, j: (n, j, 0))],
        out_specs=pl.BlockSpec((None, C, BLK), lambda n, j: (n, 0, j)),
        out_shape=jax.ShapeDtypeStruct((N, C, HW), jnp.float32),
    )(acc)


def kernel(feature_shape, all_rois_center, rois_feature_usps):
    arc = all_rois_center.reshape(LEVELS, 2, N, R, 1)
    fr = rois_feature_usps.reshape(LEVELS, N, R, C)
    w, idx = _phase_a(arc)
    wg = jnp.transpose(w.reshape(N, LEVELS, 4, G_PER, GR),
                       (0, 1, 3, 2, 4)).reshape(N, NG, CH)
    idxg = jnp.transpose(idx.reshape(N, LEVELS, 4, G_PER, GR),
                         (0, 1, 3, 2, 4)).reshape(N, NG, CH)
    zeros = jnp.zeros((RPT, CHALF), jnp.float32)
    acc = _phase_b(fr, wg, idxg, zeros)
    out = _phase_c(acc)
    return out.reshape(N, C, H, W)


# speedup vs baseline: 2.1796x; 2.1796x over previous
import functools

import jax
import jax.numpy as jnp
from jax import lax
from jax.experimental import pallas as pl
from jax.experimental.pallas import tpu as pltpu
from jax.experimental.pallas import tpu_sc as plsc

LEVELS = 4
N = 4
C = 128
H = 128
W = 128
R = 6272
RB = 784
J = R // RB

NSUB = 16
NCORE = 2
CHALF = C // NCORE
NV = CHALF // 16
HW = H * W
RPT = HW // NSUB
GR = 32
CH = 4 * GR
G_PER = R // GR
NG = LEVELS * G_PER
GPT = NG // NSUB


def _weights_body(x_ref, y_ref, w_ref, idx_ref):
    x = x_ref[...]
    y = y_ref[...]
    fx = jnp.floor(x)
    fy = jnp.floor(y)
    xp = x - fx
    yp = y - fy
    vx = xp * xp + (1.0 - xp) * (1.0 - xp)
    vy = yp * yp + (1.0 - yp) * (1.0 - yp)
    invq = 0.25 / (vx * vy)
    fxi = fx.astype(jnp.int32)
    fyi = fy.astype(jnp.int32)
    cxi = fxi + 1
    cyi = fyi + 1
    u0 = (1.0 - xp) * invq
    u1 = xp * invq
    v0 = 1.0 - yp
    v1 = yp
    bx0 = fxi < H
    bx1 = cxi < H
    by0 = fyi < H
    by1 = cyi < H
    zf = jnp.zeros_like(x)
    zi = jnp.zeros_like(fxi)
    m11 = bx0 & by0
    m12 = bx0 & by1
    m21 = bx1 & by0
    m22 = bx1 & by1
    w_ref[0, :, :] = jnp.where(m11, u0 * v0, zf)
    w_ref[1, :, :] = jnp.where(m12, u0 * v1, zf)
    w_ref[2, :, :] = jnp.where(m21, u1 * v0, zf)
    w_ref[3, :, :] = jnp.where(m22, u1 * v1, zf)
    idx_ref[0, :, :] = jnp.where(m11, fxi * W + fyi, zi)
    idx_ref[1, :, :] = jnp.where(m12, fxi * W + cyi, zi)
    idx_ref[2, :, :] = jnp.where(m21, cxi * W + fyi, zi)
    idx_ref[3, :, :] = jnp.where(m22, cxi * W + cyi, zi)


def _phase_a(arc):
    return pl.pallas_call(
        _weights_body,
        grid=(N, LEVELS),
        in_specs=[
            pl.BlockSpec((None, None, 49, 128), lambda n, l: (l, n, 0, 0)),
            pl.BlockSpec((None, None, 49, 128), lambda n, l: (l, n, 0, 0)),
        ],
        out_specs=[
            pl.BlockSpec((None, None, 4, 49, 128),
                         lambda n, l: (n, l, 0, 0, 0)),
            pl.BlockSpec((None, None, 4, 49, 128),
                         lambda n, l: (n, l, 0, 0, 0)),
        ],
        out_shape=[
            jax.ShapeDtypeStruct((N, LEVELS, 4, 49, 128), jnp.float32),
            jax.ShapeDtypeStruct((N, LEVELS, 4, 49, 128), jnp.int32),
        ],
    )(arc[:, 0], arc[:, 1])


def _sc_body(fr_hbm, w_hbm, idx_hbm, zeros_hbm, out_hbm,
             idx_all, w_all, fbufs, rbufs, fsems, ssems, acc):
    cid = lax.axis_index("c")
    sid = lax.axis_index("s")
    col0 = cid * CHALF

    for n in range(N):
        pltpu.sync_copy(zeros_hbm, acc.at[pl.ds(sid * RPT, RPT)])
        pltpu.sync_copy(idx_hbm.at[n, pl.ds(sid * GPT, GPT)], idx_all)
        pltpu.sync_copy(w_hbm.at[n, pl.ds(sid * GPT, GPT)], w_all)
        plsc.subcore_barrier()
        gbase = sid * GPT

        def feat_start(k, s):
            g = jnp.minimum(gbase + k, NG - 1)
            l = g // G_PER
            r0 = (g - l * G_PER) * GR
            pltpu.async_copy(
                fr_hbm.at[l, n, pl.ds(r0, GR), pl.ds(col0, CHALF)],
                fbufs[s], fsems[s])

        def feat_wait(s):
            pltpu.make_async_copy(
                fr_hbm.at[0, n, pl.ds(0, GR), pl.ds(col0, CHALF)],
                fbufs[s], fsems[s]).wait()

        def compute(k, s):
            fb = fbufs[s]
            rb = rbufs[s]

            def half_body(h, _):
                j0 = 16 * h
                wv = [w_all[k, pl.ds(q * GR + j0, 16)] for q in range(4)]
                for j in range(16):
                    fv = [fb[j0 + j, pl.ds(16 * c, 16)] for c in range(NV)]
                    for q in range(4):
                        wsc = wv[q][j]
                        for c in range(NV):
                            rb[q * GR + j0 + j, pl.ds(16 * c, 16)] = (
                                wsc * fv[c])
                return _

            lax.fori_loop(0, GR // 16, half_body, None)

        def scat_start(k, s):
            pltpu.async_copy(rbufs[s], acc.at[idx_all.at[k]], ssems[s],
                             add=True)

        def scat_wait(k, s):
            pltpu.make_async_copy(rbufs[s], acc.at[idx_all.at[k]],
                                  ssems[s]).wait()

        feat_start(jnp.int32(0), 0)
        feat_start(jnp.int32(1), 1)
        for k0 in range(2):
            k = jnp.int32(k0)
            feat_wait(k0)
            compute(k, k0)
            scat_start(k, k0)
            feat_start(k + 2, k0)

        def pair_body(i, _):
            for s in range(2):
                k = 2 * i + 2 + s
                feat_wait(s)
                scat_wait(k - 2, s)
                compute(k, s)
                scat_start(k, s)
                feat_start(k + 2, s)
            return _

        lax.fori_loop(0, 23, pair_body, None)
        k = jnp.int32(GPT - 1)
        feat_wait(0)
        scat_wait(k - 2, 0)
        compute(k, 0)
        scat_start(k, 0)
        feat_wait(1)
        scat_wait(k - 1, 1)
        scat_wait(k, 0)
        plsc.subcore_barrier()
        pltpu.sync_copy(
            acc.at[pl.ds(sid * RPT, RPT)],
            out_hbm.at[n, pl.ds(sid * RPT, RPT), pl.ds(col0, CHALF)])
        plsc.subcore_barrier()


def _phase_b(fr, wg, idxg, zeros):
    mesh = plsc.VectorSubcoreMesh(core_axis_name="c", subcore_axis_name="s")
    f = pl.kernel(
        _sc_body,
        out_type=jax.ShapeDtypeStruct((N, HW, C), jnp.float32),
        mesh=mesh,
        scratch_types=[
            pltpu.VMEM((GPT, CH), jnp.int32),
            pltpu.VMEM((GPT, CH), jnp.float32),
            tuple(pltpu.VMEM((GR, CHALF), jnp.float32) for _ in range(2)),
            tuple(pltpu.VMEM((CH, CHALF), jnp.float32) for _ in range(2)),
            tuple(pltpu.SemaphoreType.DMA for _ in range(2)),
            tuple(pltpu.SemaphoreType.DMA for _ in range(2)),
            pltpu.VMEM_SHARED((HW, CHALF), jnp.float32),
        ],
        compiler_params=pltpu.CompilerParams(use_tc_tiling_on_sc=False),
    )
    return f(fr, wg, idxg, zeros)


def _transpose_body(in_ref, out_ref):
    for i in range(in_ref.shape[0] // 128):
        out_ref[:, 128 * i:128 * (i + 1)] = in_ref[128 * i:128 * (i + 1), :].T


def _phase_c(acc):
    BLK = 1024
    return pl.pallas_call(
        _transpose_body,
        grid=(N, HW // BLK),
        in_specs=[pl.BlockSpec((None, BLK, C), lambda n, j: (n, j, 0))],
        out_specs=pl.BlockSpec((None, C, BLK), lambda n, j: (n, 0, j)),
        out_shape=jax.ShapeDtypeStruct((N, C, HW), jnp.float32),
    )(acc)


def kernel(feature_shape, all_rois_center, rois_feature_usps):
    arc = all_rois_center.reshape(LEVELS, 2, N, 49, 128)
    fr = rois_feature_usps.reshape(LEVELS, N, R, C)
    w, idx = _phase_a(arc)
    def _chunk_major(a):
        return jnp.transpose(a.reshape(N, LEVELS, 4, 49, 4, GR),
                             (0, 1, 3, 4, 2, 5)).reshape(N, NG, CH)

    wg = _chunk_major(w)
    idxg = _chunk_major(idx)
    zeros = jnp.zeros((RPT, CHALF), jnp.float32)
    acc = _phase_b(fr, wg, idxg, zeros)
    out = _phase_c(acc)
    return out.reshape(N, C, H, W)


# speedup vs baseline: 2.2522x; 1.0333x over previous
import functools

import jax
import jax.numpy as jnp
from jax import lax
from jax.experimental import pallas as pl
from jax.experimental.pallas import tpu as pltpu
from jax.experimental.pallas import tpu_sc as plsc

LEVELS = 4
N = 4
C = 128
H = 128
W = 128
R = 6272
RB = 784
J = R // RB

NSUB = 16
NCORE = 2
CHALF = C // NCORE
NV = CHALF // 16
HW = H * W
RPT = HW // NSUB
GR = 32
CH = 4 * GR
G_PER = R // GR
NG = LEVELS * G_PER
GPT = NG // NSUB


def _weights_body(x_ref, y_ref, w_ref, idx_ref):
    x = x_ref[...]
    y = y_ref[...]
    fx = jnp.floor(x)
    fy = jnp.floor(y)
    xp = x - fx
    yp = y - fy
    vx = xp * xp + (1.0 - xp) * (1.0 - xp)
    vy = yp * yp + (1.0 - yp) * (1.0 - yp)
    invq = 0.25 / (vx * vy)
    fxi = fx.astype(jnp.int32)
    fyi = fy.astype(jnp.int32)
    cxi = fxi + 1
    cyi = fyi + 1
    u0 = (1.0 - xp) * invq
    u1 = xp * invq
    v0 = 1.0 - yp
    v1 = yp
    bx0 = fxi < H
    bx1 = cxi < H
    by0 = fyi < H
    by1 = cyi < H
    zf = jnp.zeros_like(x)
    zi = jnp.zeros_like(fxi)
    m11 = bx0 & by0
    m12 = bx0 & by1
    m21 = bx1 & by0
    m22 = bx1 & by1
    w_ref[0, :, :] = jnp.where(m11, u0 * v0, zf)
    w_ref[1, :, :] = jnp.where(m12, u0 * v1, zf)
    w_ref[2, :, :] = jnp.where(m21, u1 * v0, zf)
    w_ref[3, :, :] = jnp.where(m22, u1 * v1, zf)
    idx_ref[0, :, :] = jnp.where(m11, fxi * W + fyi, zi)
    idx_ref[1, :, :] = jnp.where(m12, fxi * W + cyi, zi)
    idx_ref[2, :, :] = jnp.where(m21, cxi * W + fyi, zi)
    idx_ref[3, :, :] = jnp.where(m22, cxi * W + cyi, zi)


def _phase_a(arc):
    return pl.pallas_call(
        _weights_body,
        grid=(N, LEVELS),
        in_specs=[
            pl.BlockSpec((None, None, 49, 128), lambda n, l: (l, n, 0, 0)),
            pl.BlockSpec((None, None, 49, 128), lambda n, l: (l, n, 0, 0)),
        ],
        out_specs=[
            pl.BlockSpec((None, None, 4, 49, 128),
                         lambda n, l: (n, l, 0, 0, 0)),
            pl.BlockSpec((None, None, 4, 49, 128),
                         lambda n, l: (n, l, 0, 0, 0)),
        ],
        out_shape=[
            jax.ShapeDtypeStruct((N, LEVELS, 4, 49, 128), jnp.float32),
            jax.ShapeDtypeStruct((N, LEVELS, 4, 49, 128), jnp.int32),
        ],
    )(arc[:, 0], arc[:, 1])


def _sc_body(fr_hbm, w_hbm, idx_hbm, zeros_hbm, out_hbm,
             idx_all, w_all, fbufs, rbufs, fsems, ssems, acc):
    cid = lax.axis_index("c")
    sid = lax.axis_index("s")
    col0 = cid * CHALF

    def batch_body(n, _carry):
        pltpu.sync_copy(zeros_hbm, acc.at[pl.ds(sid * RPT, RPT)])
        pltpu.sync_copy(idx_hbm.at[n, pl.ds(sid * GPT, GPT)], idx_all)
        pltpu.sync_copy(w_hbm.at[n, pl.ds(sid * GPT, GPT)], w_all)
        plsc.subcore_barrier()
        gbase = sid * GPT

        def feat_start(k, s):
            g = jnp.minimum(gbase + k, NG - 1)
            l = g // G_PER
            r0 = (g - l * G_PER) * GR
            pltpu.async_copy(
                fr_hbm.at[l, n, pl.ds(r0, GR), pl.ds(col0, CHALF)],
                fbufs[s], fsems[s])

        def feat_wait(s):
            pltpu.make_async_copy(
                fr_hbm.at[0, n, pl.ds(0, GR), pl.ds(col0, CHALF)],
                fbufs[s], fsems[s]).wait()

        def compute(k, s):
            fb = fbufs[s]
            rb = rbufs[s]

            def half_body(h, _):
                j0 = 16 * h
                wv = [w_all[k, pl.ds(q * GR + j0, 16)] for q in range(4)]
                for j in range(16):
                    fv = [fb[j0 + j, pl.ds(16 * c, 16)] for c in range(NV)]
                    for q in range(4):
                        wsc = wv[q][j]
                        for c in range(NV):
                            rb[q * GR + j0 + j, pl.ds(16 * c, 16)] = (
                                wsc * fv[c])
                return _

            lax.fori_loop(0, GR // 16, half_body, None)

        def scat_start(k, s):
            pltpu.async_copy(rbufs[s], acc.at[idx_all.at[k]], ssems[s],
                             add=True)

        def scat_wait(k, s):
            pltpu.make_async_copy(rbufs[s], acc.at[idx_all.at[k]],
                                  ssems[s]).wait()

        for k0 in range(3):
            feat_start(jnp.int32(k0), k0)
        for k0 in range(3):
            k = jnp.int32(k0)
            feat_wait(k0)
            compute(k, k0)
            scat_start(k, k0)
            feat_start(k + 3, k0)

        def tri_body(i, _):
            for s in range(3):
                k = 3 * i + 3 + s
                feat_wait(s)
                scat_wait(k - 3, s)
                compute(k, s)
                scat_start(k, s)
                feat_start(k + 3, s)
            return _

        lax.fori_loop(0, 15, tri_body, None)
        k = jnp.int32(GPT - 1)
        feat_wait(0)
        scat_wait(k - 3, 0)
        compute(k, 0)
        scat_start(k, 0)
        feat_wait(1)
        feat_wait(2)
        scat_wait(k - 2, 1)
        scat_wait(k - 1, 2)
        scat_wait(k, 0)
        plsc.subcore_barrier()
        pltpu.sync_copy(
            acc.at[pl.ds(sid * RPT, RPT)],
            out_hbm.at[n, pl.ds(sid * RPT, RPT), pl.ds(col0, CHALF)])
        plsc.subcore_barrier()
        return _carry

    lax.fori_loop(0, N, batch_body, None)


def _phase_b(fr, wg, idxg, zeros):
    mesh = plsc.VectorSubcoreMesh(core_axis_name="c", subcore_axis_name="s")
    f = pl.kernel(
        _sc_body,
        out_type=jax.ShapeDtypeStruct((N, HW, C), jnp.float32),
        mesh=mesh,
        scratch_types=[
            pltpu.VMEM((GPT, CH), jnp.int32),
            pltpu.VMEM((GPT, CH), jnp.float32),
            tuple(pltpu.VMEM((GR, CHALF), jnp.float32) for _ in range(3)),
            tuple(pltpu.VMEM((CH, CHALF), jnp.float32) for _ in range(3)),
            tuple(pltpu.SemaphoreType.DMA for _ in range(3)),
            tuple(pltpu.SemaphoreType.DMA for _ in range(3)),
            pltpu.VMEM_SHARED((HW, CHALF), jnp.float32),
        ],
        compiler_params=pltpu.CompilerParams(use_tc_tiling_on_sc=False),
    )
    return f(fr, wg, idxg, zeros)


def _transpose_body(in_ref, out_ref):
    for i in range(in_ref.shape[0] // 128):
        out_ref[:, 128 * i:128 * (i + 1)] = in_ref[128 * i:128 * (i + 1), :].T


def _phase_c(acc):
    BLK = 1024
    return pl.pallas_call(
        _transpose_body,
        grid=(N, HW // BLK),
        in_specs=[pl.BlockSpec((None, BLK, C), lambda n, j: (n, j, 0))],
        out_specs=pl.BlockSpec((None, C, BLK), lambda n, j: (n, 0, j)),
        out_shape=jax.ShapeDtypeStruct((N, C, HW), jnp.float32),
    )(acc)


def kernel(feature_shape, all_rois_center, rois_feature_usps):
    arc = all_rois_center.reshape(LEVELS, 2, N, 49, 128)
    fr = rois_feature_usps.reshape(LEVELS, N, R, C)
    w, idx = _phase_a(arc)
    def _chunk_major(a):
        return jnp.transpose(a.reshape(N, LEVELS, 4, 49, 4, GR),
                             (0, 1, 3, 4, 2, 5)).reshape(N, NG, CH)

    wg = _chunk_major(w)
    idxg = _chunk_major(idx)
    zeros = jnp.zeros((RPT, CHALF), jnp.float32)
    acc = _phase_b(fr, wg, idxg, zeros)
    out = _phase_c(acc)
    return out.reshape(N, C, H, W)
